# initial kernel scaffold (unmeasured)
import jax
import jax.numpy as jnp
from jax import lax
from jax.experimental import pallas as pl
from jax.experimental.pallas import tpu as pltpu

N_DEV = 4
BLK = 64
STRIDE = 4


def kernel(x, Wq, K_ext, V_ext, Wo):
    B, S, D = x.shape
    Hq, Dh = K_ext.shape[2], K_ext.shape[3]
    HD = Hq * Dh

    x = x.astype(jnp.bfloat16)
    Wq = Wq.astype(jnp.bfloat16)
    Wo = Wo.astype(jnp.bfloat16)
    K2 = K_ext.reshape(B, S, HD).astype(jnp.bfloat16)
    V2 = V_ext.reshape(B, S, HD).astype(jnp.bfloat16)

    def body(x_ref, wq_ref, k_ref, v_ref, wo_ref, out_ref,
             k_bufs, v_bufs, q_sc, ctx_sc,
             ksend, krecv, vsend, vrecv):
        my = lax.axis_index("i")
        left = lax.rem(my + N_DEV - 1, N_DEV)
        right = lax.rem(my + 1, N_DEV)

        barrier = pltpu.get_barrier_semaphore()
        for nbr in (left, right):
            pl.semaphore_signal(barrier, inc=1, device_id=(nbr,),
                                device_id_type=pl.DeviceIdType.MESH)
        pl.semaphore_wait(barrier, 2)

        for hp in range(N_DEV - 1):
            ksrc = k_ref if hp == 0 else k_bufs.at[hp - 1]
            vsrc = v_ref if hp == 0 else v_bufs.at[hp - 1]
            rk = pltpu.make_async_remote_copy(
                src_ref=ksrc, dst_ref=k_bufs.at[hp],
                send_sem=ksend.at[hp], recv_sem=krecv.at[hp],
                device_id=(right,), device_id_type=pl.DeviceIdType.MESH)
            rv = pltpu.make_async_remote_copy(
                src_ref=vsrc, dst_ref=v_bufs.at[hp],
                send_sem=vsend.at[hp], recv_sem=vrecv.at[hp],
                device_id=(right,), device_id_type=pl.DeviceIdType.MESH)
            rk.start()
            rv.start()
            rk.wait()
            rv.wait()

        for b in range(B):
            q_sc[b] = lax.dot_general(
                x_ref[b], wq_ref[...], (((1,), (0,)), ((), ())),
                preferred_element_type=jnp.float32).astype(jnp.bfloat16)

        rows = lax.broadcasted_iota(jnp.int32, (S, S), 0)
        cols = lax.broadcasted_iota(jnp.int32, (S, S), 1)
        mask = ((rows // BLK) % STRIDE) == ((cols // BLK) % STRIDE)

        for b in range(B):
            for h in range(Hq):
                sl = slice(h * Dh, (h + 1) * Dh)
                qbh = q_sc[b, :, sl]
                ss = []
                for a in range(N_DEV):
                    kc = k_ref[b, :, sl] if a == 0 else k_bufs[a - 1, b, :, sl]
                    sa = lax.dot_general(
                        qbh, kc, (((1,), (1,)), ((), ())),
                        preferred_element_type=jnp.float32) * 0.125
                    ss.append(jnp.where(mask, sa, -1e9))
                m = ss[0]
                for sa in ss[1:]:
                    m = jnp.maximum(m, sa)
                mrow = jnp.max(m, axis=1, keepdims=True)
                acc = None
                lsum = None
                for a in range(N_DEV):
                    w = jnp.exp(ss[a] - mrow)
                    vc = v_ref[b, :, sl] if a == 0 else v_bufs[a - 1, b, :, sl]
                    pa = lax.dot_general(
                        w.astype(jnp.bfloat16), vc, (((1,), (0,)), ((), ())),
                        preferred_element_type=jnp.float32)
                    acc = pa if acc is None else acc + pa
                    wsum = jnp.sum(w, axis=1, keepdims=True)
                    lsum = wsum if lsum is None else lsum + wsum
                ctx_sc[b, :, sl] = (acc / lsum).astype(jnp.bfloat16)

        for b in range(B):
            out_ref[b] = lax.dot_general(
                ctx_sc[b], wo_ref[...], (((1,), (0,)), ((), ())),
                preferred_element_type=jnp.float32)

    return pl.pallas_call(
        body,
        out_shape=jax.ShapeDtypeStruct((B, S, D), jnp.float32),
        in_specs=[pl.BlockSpec(memory_space=pltpu.VMEM)] * 5,
        out_specs=pl.BlockSpec(memory_space=pltpu.VMEM),
        scratch_shapes=[
            pltpu.VMEM((N_DEV - 1, B, S, HD), jnp.bfloat16),
            pltpu.VMEM((N_DEV - 1, B, S, HD), jnp.bfloat16),
            pltpu.VMEM((B, S, HD), jnp.bfloat16),
            pltpu.VMEM((B, S, HD), jnp.bfloat16),
            pltpu.SemaphoreType.DMA((N_DEV - 1,)),
            pltpu.SemaphoreType.DMA((N_DEV - 1,)),
            pltpu.SemaphoreType.DMA((N_DEV - 1,)),
            pltpu.SemaphoreType.DMA((N_DEV - 1,)),
        ],
        compiler_params=pltpu.CompilerParams(collective_id=0),
    )(x, Wq, K2, V2, Wo)


# baseline (device time: 108478 ns/iter reference)
import jax
import jax.numpy as jnp
from jax import lax
from jax.experimental import pallas as pl
from jax.experimental.pallas import tpu as pltpu

N_DEV = 4
BLK = 64
STRIDE = 4


def kernel(x, Wq, K_ext, V_ext, Wo):
    B, S, D = x.shape
    Hq, Dh = K_ext.shape[2], K_ext.shape[3]
    HD = Hq * Dh

    x = x.astype(jnp.bfloat16)
    Wq = Wq.astype(jnp.bfloat16)
    Wo = Wo.astype(jnp.bfloat16)
    K2 = K_ext.reshape(B, S, HD).astype(jnp.bfloat16)
    V2 = V_ext.reshape(B, S, HD).astype(jnp.bfloat16)

    def body(x_ref, wq_ref, k_ref, v_ref, wo_ref, out_ref,
             k_bufs, v_bufs, q_sc,
             ksend, krecv, vsend, vrecv):
        my = lax.axis_index("i")
        left = lax.rem(my + N_DEV - 1, N_DEV)
        right = lax.rem(my + 1, N_DEV)

        barrier = pltpu.get_barrier_semaphore()
        for nbr in (left, right):
            pl.semaphore_signal(barrier, inc=1, device_id=(nbr,),
                                device_id_type=pl.DeviceIdType.MESH)
        pl.semaphore_wait(barrier, 2)

        for hp in range(N_DEV - 1):
            ksrc = k_ref if hp == 0 else k_bufs.at[hp - 1]
            vsrc = v_ref if hp == 0 else v_bufs.at[hp - 1]
            rk = pltpu.make_async_remote_copy(
                src_ref=ksrc, dst_ref=k_bufs.at[hp],
                send_sem=ksend.at[hp], recv_sem=krecv.at[hp],
                device_id=(right,), device_id_type=pl.DeviceIdType.MESH)
            rv = pltpu.make_async_remote_copy(
                src_ref=vsrc, dst_ref=v_bufs.at[hp],
                send_sem=vsend.at[hp], recv_sem=vrecv.at[hp],
                device_id=(right,), device_id_type=pl.DeviceIdType.MESH)
            rk.start()
            rv.start()
            rk.wait()
            rv.wait()

        for b in range(B):
            q_sc[b] = lax.dot_general(
                x_ref[b], wq_ref[...], (((1,), (0,)), ((), ())),
                preferred_element_type=jnp.float32).astype(jnp.bfloat16)

        rows = lax.broadcasted_iota(jnp.int32, (S, S), 0)
        cols = lax.broadcasted_iota(jnp.int32, (S, S), 1)
        mask = ((rows // BLK) % STRIDE) == ((cols // BLK) % STRIDE)

        for b in range(B):
            for h in range(Hq):
                sl = slice(h * Dh, (h + 1) * Dh)
                qbh = q_sc[b, :, sl]
                acc = jnp.zeros((S, Dh), jnp.float32)
                lsum = jnp.zeros((S, 1), jnp.float32)
                for a in range(N_DEV):
                    kc = k_ref[b, :, sl] if a == 0 else k_bufs[a - 1, b, :, sl]
                    s = lax.dot_general(
                        qbh, kc, (((1,), (1,)), ((), ())),
                        preferred_element_type=jnp.float32) * 0.125
                    w = jnp.where(mask, jnp.exp(s), 0.0)
                    vc = v_ref[b, :, sl] if a == 0 else v_bufs[a - 1, b, :, sl]
                    acc = acc + lax.dot_general(
                        w.astype(jnp.bfloat16), vc, (((1,), (0,)), ((), ())),
                        preferred_element_type=jnp.float32)
                    lsum = lsum + jnp.sum(w, axis=1, keepdims=True)
                q_sc[b, :, sl] = (acc / lsum).astype(jnp.bfloat16)

        for b in range(B):
            out_ref[b] = lax.dot_general(
                q_sc[b], wo_ref[...], (((1,), (0,)), ((), ())),
                preferred_element_type=jnp.float32).astype(jnp.bfloat16)

    out = pl.pallas_call(
        body,
        out_shape=jax.ShapeDtypeStruct((B, S, D), jnp.bfloat16),
        in_specs=[pl.BlockSpec(memory_space=pltpu.VMEM)] * 5,
        out_specs=pl.BlockSpec(memory_space=pltpu.VMEM),
        scratch_shapes=[
            pltpu.VMEM((N_DEV - 1, B, S, HD), jnp.bfloat16),
            pltpu.VMEM((N_DEV - 1, B, S, HD), jnp.bfloat16),
            pltpu.VMEM((B, S, HD), jnp.bfloat16),
            pltpu.SemaphoreType.DMA((N_DEV - 1,)),
            pltpu.SemaphoreType.DMA((N_DEV - 1,)),
            pltpu.SemaphoreType.DMA((N_DEV - 1,)),
            pltpu.SemaphoreType.DMA((N_DEV - 1,)),
        ],
        compiler_params=pltpu.CompilerParams(collective_id=0),
    )(x, Wq, K2, V2, Wo)
    return out.astype(jnp.float32)


# device time: 56506 ns/iter; 1.9198x vs baseline; 1.9198x over previous
import jax
import jax.numpy as jnp
from jax import lax
from jax.experimental import pallas as pl
from jax.experimental.pallas import tpu as pltpu

N_DEV = 4
BLK = 64
STRIDE = 4


def kernel(x, Wq, K_ext, V_ext, Wo):
    B, S, D = x.shape
    Hq, Dh = K_ext.shape[2], K_ext.shape[3]
    HD = Hq * Dh

    x = x.astype(jnp.bfloat16)
    Wq = Wq.astype(jnp.bfloat16)
    Wo = Wo.astype(jnp.bfloat16)
    K2 = K_ext.reshape(B, S, HD).astype(jnp.bfloat16)
    V2 = V_ext.reshape(B, S, HD).astype(jnp.bfloat16)

    def body(x_ref, wq_ref, k_ref, v_ref, wo_ref, out_ref,
             k_bufs, v_bufs, q_sc,
             ksend, krecv, vsend, vrecv):
        my = lax.axis_index("i")
        left = lax.rem(my + N_DEV - 1, N_DEV)
        right = lax.rem(my + 1, N_DEV)

        barrier = pltpu.get_barrier_semaphore()
        for nbr in (left, right):
            pl.semaphore_signal(barrier, inc=1, device_id=(nbr,),
                                device_id_type=pl.DeviceIdType.MESH)
        pl.semaphore_wait(barrier, 2)

        def make_hop(hp):
            rds = []
            for d, tgt in ((0, right), (1, left)):
                for ref_, bufs, ssem, rsem in (
                        (k_ref, k_bufs, ksend, krecv),
                        (v_ref, v_bufs, vsend, vrecv)):
                    src = ref_.at[d] if hp == 0 else bufs.at[hp - 1, d]
                    rds.append(pltpu.make_async_remote_copy(
                        src_ref=src, dst_ref=bufs.at[hp, d],
                        send_sem=ssem.at[hp, d], recv_sem=rsem.at[hp, d],
                        device_id=(tgt,),
                        device_id_type=pl.DeviceIdType.MESH))
            return rds

        hops = [make_hop(0)]
        for r in hops[0]:
            r.start()

        for b in range(B):
            q_sc[b] = lax.dot_general(
                x_ref[b], wq_ref[...], (((1,), (0,)), ((), ())),
                preferred_element_type=jnp.float32).astype(jnp.bfloat16)

        rows = lax.broadcasted_iota(jnp.int32, (S, S), 0)
        cols = lax.broadcasted_iota(jnp.int32, (S, S), 1)
        mask = ((rows // BLK) % STRIDE) == ((cols // BLK) % STRIDE)

        accs = [[jnp.zeros((S, Dh), jnp.float32) for _ in range(Hq)]
                for _ in range(B)]
        lsums = [[jnp.zeros((S, 1), jnp.float32) for _ in range(Hq)]
                 for _ in range(B)]

        for a in range(N_DEV):
            if a >= 1:
                for r in hops[a - 1]:
                    r.wait_recv()
                if a <= N_DEV - 2:
                    hop = make_hop(a)
                    for r in hop:
                        r.start()
                    hops.append(hop)
            for b in range(B):
                for h in range(Hq):
                    sl = slice(h * Dh, (h + 1) * Dh)
                    qbh = q_sc[b, :, sl]
                    kc = k_ref[b, :, sl] if a == 0 else k_bufs[a - 1, b, :, sl]
                    s = lax.dot_general(
                        qbh, kc, (((1,), (1,)), ((), ())),
                        preferred_element_type=jnp.float32) * 0.125
                    w = jnp.where(mask, jnp.exp(s), 0.0)
                    vc = v_ref[b, :, sl] if a == 0 else v_bufs[a - 1, b, :, sl]
                    accs[b][h] = accs[b][h] + lax.dot_general(
                        w.astype(jnp.bfloat16), vc, (((1,), (0,)), ((), ())),
                        preferred_element_type=jnp.float32)
                    lsums[b][h] = lsums[b][h] + jnp.sum(w, axis=1,
                                                        keepdims=True)

        for b in range(B):
            for h in range(Hq):
                sl = slice(h * Dh, (h + 1) * Dh)
                q_sc[b, :, sl] = (accs[b][h] / lsums[b][h]).astype(jnp.bfloat16)

        for b in range(B):
            out_ref[b] = lax.dot_general(
                q_sc[b], wo_ref[...], (((1,), (0,)), ((), ())),
                preferred_element_type=jnp.float32).astype(jnp.bfloat16)

        for hop in hops:
            for r in hop:
                r.wait_send()

    out = pl.pallas_call(
        body,
        out_shape=jax.ShapeDtypeStruct((B, S, D), jnp.bfloat16),
        in_specs=[pl.BlockSpec(memory_space=pltpu.VMEM)] * 5,
        out_specs=pl.BlockSpec(memory_space=pltpu.VMEM),
        scratch_shapes=[
            pltpu.VMEM((N_DEV - 1, B, S, HD), jnp.bfloat16),
            pltpu.VMEM((N_DEV - 1, B, S, HD), jnp.bfloat16),
            pltpu.VMEM((B, S, HD), jnp.bfloat16),
            pltpu.SemaphoreType.DMA((N_DEV - 1, 2)),
            pltpu.SemaphoreType.DMA((N_DEV - 1, 2)),
            pltpu.SemaphoreType.DMA((N_DEV - 1, 2)),
            pltpu.SemaphoreType.DMA((N_DEV - 1, 2)),
        ],
        compiler_params=pltpu.CompilerParams(collective_id=0),
    )(x, Wq, K2, V2, Wo)
    return out.astype(jnp.float32)


# device time: 53129 ns/iter; 2.0418x vs baseline; 1.0636x over previous
import jax
import jax.numpy as jnp
from jax import lax
from jax.experimental import pallas as pl
from jax.experimental.pallas import tpu as pltpu

N_DEV = 4
BLK = 64
STRIDE = 4


def kernel(x, Wq, K_ext, V_ext, Wo):
    B, S, D = x.shape
    Hq, Dh = K_ext.shape[2], K_ext.shape[3]
    HD = Hq * Dh

    x = x.astype(jnp.bfloat16)
    Wq = Wq.astype(jnp.bfloat16)
    Wo = Wo.astype(jnp.bfloat16)
    K2 = K_ext.reshape(B, S, HD).astype(jnp.bfloat16)
    V2 = V_ext.reshape(B, S, HD).astype(jnp.bfloat16)

    def body(x_ref, wq_ref, k_ref, v_ref, wo_ref, out_ref,
             k_bufs, v_bufs, q_sc,
             ksend, krecv, vsend, vrecv):
        my = lax.axis_index("i")
        left = lax.rem(my + N_DEV - 1, N_DEV)
        right = lax.rem(my + 1, N_DEV)

        barrier = pltpu.get_barrier_semaphore()
        for nbr in (left, right):
            pl.semaphore_signal(barrier, inc=1, device_id=(nbr,),
                                device_id_type=pl.DeviceIdType.MESH)
        pl.semaphore_wait(barrier, 2)

        def make_rdmas(slot, dirs, src_is_input):
            rds = []
            for d, tgt in dirs:
                for ref_, bufs, ssem, rsem in (
                        (k_ref, k_bufs, ksend, krecv),
                        (v_ref, v_bufs, vsend, vrecv)):
                    src = ref_.at[d] if src_is_input else bufs.at[slot - 1, d]
                    rds.append(pltpu.make_async_remote_copy(
                        src_ref=src, dst_ref=bufs.at[slot, d],
                        send_sem=ssem.at[slot, d], recv_sem=rsem.at[slot, d],
                        device_id=(tgt,),
                        device_id_type=pl.DeviceIdType.MESH))
            return rds

        hop0 = make_rdmas(0, ((0, right), (1, left)), True)
        direct = make_rdmas(2, ((0, left), (1, right)), True)
        for r in hop0:
            r.start()
        for r in direct:
            r.start()

        for b in range(B):
            q_sc[b] = lax.dot_general(
                x_ref[b], wq_ref[...], (((1,), (0,)), ((), ())),
                preferred_element_type=jnp.float32).astype(jnp.bfloat16)

        rows = lax.broadcasted_iota(jnp.int32, (S, S), 0)
        cols = lax.broadcasted_iota(jnp.int32, (S, S), 1)
        mask = ((rows // BLK) % STRIDE) == ((cols // BLK) % STRIDE)

        accs = [[jnp.zeros((S, Dh), jnp.float32) for _ in range(Hq)]
                for _ in range(B)]
        lsums = [[jnp.zeros((S, 1), jnp.float32) for _ in range(Hq)]
                 for _ in range(B)]

        def accumulate(slot):
            for b in range(B):
                for h in range(Hq):
                    sl = slice(h * Dh, (h + 1) * Dh)
                    qbh = q_sc[b, :, sl]
                    if slot is None:
                        kc = k_ref[b, :, sl]
                        vc = v_ref[b, :, sl]
                    else:
                        kc = k_bufs[slot, b, :, sl]
                        vc = v_bufs[slot, b, :, sl]
                    s = lax.dot_general(
                        qbh, kc, (((1,), (1,)), ((), ())),
                        preferred_element_type=jnp.float32) * 0.125
                    w = jnp.where(mask, jnp.exp(s), 0.0)
                    accs[b][h] = accs[b][h] + lax.dot_general(
                        w.astype(jnp.bfloat16), vc, (((1,), (0,)), ((), ())),
                        preferred_element_type=jnp.float32)
                    lsums[b][h] = lsums[b][h] + jnp.sum(w, axis=1,
                                                        keepdims=True)

        accumulate(None)

        for r in hop0:
            r.wait_recv()
        hop1 = make_rdmas(1, ((0, right), (1, left)), False)
        for r in hop1:
            r.start()
        accumulate(0)

        for r in direct:
            r.wait_recv()
        accumulate(2)

        for r in hop1:
            r.wait_recv()
        accumulate(1)

        for b in range(B):
            for h in range(Hq):
                sl = slice(h * Dh, (h + 1) * Dh)
                q_sc[b, :, sl] = (accs[b][h] / lsums[b][h]).astype(jnp.bfloat16)

        for b in range(B):
            out_ref[b] = lax.dot_general(
                q_sc[b], wo_ref[...], (((1,), (0,)), ((), ())),
                preferred_element_type=jnp.float32).astype(jnp.bfloat16)

        for r in hop0 + direct + hop1:
            r.wait_send()

    out = pl.pallas_call(
        body,
        out_shape=jax.ShapeDtypeStruct((B, S, D), jnp.bfloat16),
        in_specs=[pl.BlockSpec(memory_space=pltpu.VMEM)] * 5,
        out_specs=pl.BlockSpec(memory_space=pltpu.VMEM),
        scratch_shapes=[
            pltpu.VMEM((N_DEV - 1, B, S, HD), jnp.bfloat16),
            pltpu.VMEM((N_DEV - 1, B, S, HD), jnp.bfloat16),
            pltpu.VMEM((B, S, HD), jnp.bfloat16),
            pltpu.SemaphoreType.DMA((N_DEV - 1, 2)),
            pltpu.SemaphoreType.DMA((N_DEV - 1, 2)),
            pltpu.SemaphoreType.DMA((N_DEV - 1, 2)),
            pltpu.SemaphoreType.DMA((N_DEV - 1, 2)),
        ],
        compiler_params=pltpu.CompilerParams(collective_id=0),
    )(x, Wq, K2, V2, Wo)
    return out.astype(jnp.float32)


# device time: 51929 ns/iter; 2.0890x vs baseline; 1.0231x over previous
import jax
import jax.numpy as jnp
from jax import lax
from jax.experimental import pallas as pl
from jax.experimental.pallas import tpu as pltpu

N_DEV = 4
BLK = 64
STRIDE = 4


def kernel(x, Wq, K_ext, V_ext, Wo):
    B, S, D = x.shape
    Hq, Dh = K_ext.shape[2], K_ext.shape[3]
    HD = Hq * Dh

    K2 = K_ext.reshape(B, S, HD)
    V2 = V_ext.reshape(B, S, HD)

    def body(x_ref, wq_ref, k_ref, v_ref, wo_ref, out_ref,
             k_loc, v_loc, k_bufs, v_bufs, q_sc,
             ksend, krecv, vsend, vrecv):
        my = lax.axis_index("i")
        left = lax.rem(my + N_DEV - 1, N_DEV)
        right = lax.rem(my + 1, N_DEV)

        k_loc[...] = k_ref[...].astype(jnp.bfloat16)
        v_loc[...] = v_ref[...].astype(jnp.bfloat16)

        barrier = pltpu.get_barrier_semaphore()
        for nbr in (left, right):
            pl.semaphore_signal(barrier, inc=1, device_id=(nbr,),
                                device_id_type=pl.DeviceIdType.MESH)
        pl.semaphore_wait(barrier, 2)

        def make_rdmas(slot, dirs, src_is_input):
            rds = []
            for d, tgt in dirs:
                for loc, bufs, ssem, rsem in (
                        (k_loc, k_bufs, ksend, krecv),
                        (v_loc, v_bufs, vsend, vrecv)):
                    src = loc.at[d] if src_is_input else bufs.at[slot - 1, d]
                    rds.append(pltpu.make_async_remote_copy(
                        src_ref=src, dst_ref=bufs.at[slot, d],
                        send_sem=ssem.at[slot, d], recv_sem=rsem.at[slot, d],
                        device_id=(tgt,),
                        device_id_type=pl.DeviceIdType.MESH))
            return rds

        hop0 = make_rdmas(0, ((0, right), (1, left)), True)
        direct = make_rdmas(2, ((0, left), (1, right)), True)
        for r in hop0:
            r.start()
        for r in direct:
            r.start()

        for b in range(B):
            q_sc[b] = lax.dot_general(
                x_ref[b].astype(jnp.bfloat16),
                wq_ref[...].astype(jnp.bfloat16),
                (((1,), (0,)), ((), ())),
                preferred_element_type=jnp.float32).astype(jnp.bfloat16)

        rows = lax.broadcasted_iota(jnp.int32, (S, S), 0)
        cols = lax.broadcasted_iota(jnp.int32, (S, S), 1)
        mask = ((rows // BLK) % STRIDE) == ((cols // BLK) % STRIDE)

        accs = [[jnp.zeros((S, Dh), jnp.float32) for _ in range(Hq)]
                for _ in range(B)]
        lsums = [[jnp.zeros((S, 1), jnp.float32) for _ in range(Hq)]
                 for _ in range(B)]

        def accumulate(slot):
            for b in range(B):
                for h in range(Hq):
                    sl = slice(h * Dh, (h + 1) * Dh)
                    qbh = q_sc[b, :, sl]
                    if slot is None:
                        kc = k_loc[b, :, sl]
                        vc = v_loc[b, :, sl]
                    else:
                        kc = k_bufs[slot, b, :, sl]
                        vc = v_bufs[slot, b, :, sl]
                    s = lax.dot_general(
                        qbh, kc, (((1,), (1,)), ((), ())),
                        preferred_element_type=jnp.float32) * 0.125
                    w = jnp.where(mask, jnp.exp(s), 0.0)
                    accs[b][h] = accs[b][h] + lax.dot_general(
                        w.astype(jnp.bfloat16), vc, (((1,), (0,)), ((), ())),
                        preferred_element_type=jnp.float32)
                    lsums[b][h] = lsums[b][h] + jnp.sum(w, axis=1,
                                                        keepdims=True)

        accumulate(None)

        for r in hop0:
            r.wait_recv()
        hop1 = make_rdmas(1, ((0, right), (1, left)), False)
        for r in hop1:
            r.start()
        accumulate(0)

        for r in direct:
            r.wait_recv()
        accumulate(2)

        for r in hop1:
            r.wait_recv()
        accumulate(1)

        for b in range(B):
            for h in range(Hq):
                sl = slice(h * Dh, (h + 1) * Dh)
                q_sc[b, :, sl] = (accs[b][h] / lsums[b][h]).astype(jnp.bfloat16)

        wo_bf = wo_ref[...].astype(jnp.bfloat16)
        for b in range(B):
            out_ref[b] = lax.dot_general(
                q_sc[b], wo_bf, (((1,), (0,)), ((), ())),
                preferred_element_type=jnp.float32).astype(jnp.bfloat16)

        for r in hop0 + direct + hop1:
            r.wait_send()

    out = pl.pallas_call(
        body,
        out_shape=jax.ShapeDtypeStruct((B, S, D), jnp.bfloat16),
        in_specs=[pl.BlockSpec(memory_space=pltpu.VMEM)] * 5,
        out_specs=pl.BlockSpec(memory_space=pltpu.VMEM),
        scratch_shapes=[
            pltpu.VMEM((B, S, HD), jnp.bfloat16),
            pltpu.VMEM((B, S, HD), jnp.bfloat16),
            pltpu.VMEM((N_DEV - 1, B, S, HD), jnp.bfloat16),
            pltpu.VMEM((N_DEV - 1, B, S, HD), jnp.bfloat16),
            pltpu.VMEM((B, S, HD), jnp.bfloat16),
            pltpu.SemaphoreType.DMA((N_DEV - 1, 2)),
            pltpu.SemaphoreType.DMA((N_DEV - 1, 2)),
            pltpu.SemaphoreType.DMA((N_DEV - 1, 2)),
            pltpu.SemaphoreType.DMA((N_DEV - 1, 2)),
        ],
        compiler_params=pltpu.CompilerParams(collective_id=0),
    )(x, Wq, K2, V2, Wo)
    return out.astype(jnp.float32)


# device time: 49441 ns/iter; 2.1941x vs baseline; 1.0503x over previous
import jax
import jax.numpy as jnp
from jax import lax
from jax.experimental import pallas as pl
from jax.experimental.pallas import tpu as pltpu

N_DEV = 4
BLK = 64
STRIDE = 4


def kernel(x, Wq, K_ext, V_ext, Wo):
    B, S, D = x.shape
    Hq, Dh = K_ext.shape[2], K_ext.shape[3]
    HD = Hq * Dh

    K2 = K_ext.reshape(B, S, HD)
    V2 = V_ext.reshape(B, S, HD)

    def body(x_ref, wq_ref, k_ref, v_ref, wo_ref, out_ref,
             k_loc, v_loc, k_bufs, v_bufs, q_sc,
             ksend, krecv, vsend, vrecv):
        my = lax.axis_index("i")
        left = lax.rem(my + N_DEV - 1, N_DEV)
        right = lax.rem(my + 1, N_DEV)

        k_loc[...] = k_ref[...].astype(jnp.bfloat16)

        barrier = pltpu.get_barrier_semaphore()
        for nbr in (left, right):
            pl.semaphore_signal(barrier, inc=1, device_id=(nbr,),
                                device_id_type=pl.DeviceIdType.MESH)
        pl.semaphore_wait(barrier, 2)

        def make_rdmas(loc, bufs, ssem, rsem, slot, dirs, src_is_input):
            rds = []
            for d, tgt in dirs:
                src = loc.at[d] if src_is_input else bufs.at[slot - 1, d]
                rds.append(pltpu.make_async_remote_copy(
                    src_ref=src, dst_ref=bufs.at[slot, d],
                    send_sem=ssem.at[slot, d], recv_sem=rsem.at[slot, d],
                    device_id=(tgt,),
                    device_id_type=pl.DeviceIdType.MESH))
            return rds

        RING = ((0, right), (1, left))
        REV = ((0, left), (1, right))

        def k_rdmas(slot, dirs, src_is_input=False):
            return make_rdmas(k_loc, k_bufs, ksend, krecv, slot, dirs,
                              src_is_input)

        def v_rdmas(slot, dirs, src_is_input=False):
            return make_rdmas(v_loc, v_bufs, vsend, vrecv, slot, dirs,
                              src_is_input)

        hop0_k = k_rdmas(0, RING, True)
        direct_k = k_rdmas(2, REV, True)
        for r in hop0_k + direct_k:
            r.start()
        v_loc[...] = v_ref[...].astype(jnp.bfloat16)
        hop0_v = v_rdmas(0, RING, True)
        direct_v = v_rdmas(2, REV, True)
        for r in hop0_v + direct_v:
            r.start()

        for b in range(B):
            q_sc[b] = lax.dot_general(
                x_ref[b].astype(jnp.bfloat16),
                wq_ref[...].astype(jnp.bfloat16),
                (((1,), (0,)), ((), ())),
                preferred_element_type=jnp.float32).astype(jnp.bfloat16)

        rows = lax.broadcasted_iota(jnp.int32, (S, S), 0)
        cols = lax.broadcasted_iota(jnp.int32, (S, S), 1)
        mask = ((rows // BLK) % STRIDE) == ((cols // BLK) % STRIDE)

        accs = [[jnp.zeros((S, Dh), jnp.float32) for _ in range(Hq)]
                for _ in range(B)]
        lsums = [[jnp.zeros((S, 1), jnp.float32) for _ in range(Hq)]
                 for _ in range(B)]

        def weights(slot, b, h):
            sl = slice(h * Dh, (h + 1) * Dh)
            qbh = q_sc[b, :, sl]
            kc = k_loc[b, :, sl] if slot is None else k_bufs[slot, b, :, sl]
            s = lax.dot_general(
                qbh, kc, (((1,), (1,)), ((), ())),
                preferred_element_type=jnp.float32) * 0.125
            return jnp.where(mask, jnp.exp(s), 0.0)

        def pv(slot, b, h, w):
            sl = slice(h * Dh, (h + 1) * Dh)
            vc = v_loc[b, :, sl] if slot is None else v_bufs[slot, b, :, sl]
            accs[b][h] = accs[b][h] + lax.dot_general(
                w.astype(jnp.bfloat16), vc, (((1,), (0,)), ((), ())),
                preferred_element_type=jnp.float32)
            lsums[b][h] = lsums[b][h] + jnp.sum(w, axis=1, keepdims=True)

        def accumulate(slot):
            for b in range(B):
                for h in range(Hq):
                    pv(slot, b, h, weights(slot, b, h))

        accumulate(None)

        for r in hop0_k + hop0_v:
            r.wait_recv()
        hop1_k = k_rdmas(1, RING)
        hop1_v = v_rdmas(1, RING)
        for r in hop1_k + hop1_v:
            r.start()
        accumulate(0)

        for r in direct_k + direct_v:
            r.wait_recv()
        accumulate(2)

        for r in hop1_k:
            r.wait_recv()
        ws = [[weights(1, b, h) for h in range(Hq)] for b in range(B)]
        for r in hop1_v:
            r.wait_recv()
        for b in range(B):
            for h in range(Hq):
                pv(1, b, h, ws[b][h])

        for b in range(B):
            for h in range(Hq):
                sl = slice(h * Dh, (h + 1) * Dh)
                q_sc[b, :, sl] = (accs[b][h] / lsums[b][h]).astype(jnp.bfloat16)

        wo_bf = wo_ref[...].astype(jnp.bfloat16)
        for b in range(B):
            out_ref[b] = lax.dot_general(
                q_sc[b], wo_bf, (((1,), (0,)), ((), ())),
                preferred_element_type=jnp.float32).astype(jnp.bfloat16)

        for r in (hop0_k + hop0_v + direct_k + direct_v + hop1_k + hop1_v):
            r.wait_send()

    out = pl.pallas_call(
        body,
        out_shape=jax.ShapeDtypeStruct((B, S, D), jnp.bfloat16),
        in_specs=[pl.BlockSpec(memory_space=pltpu.VMEM)] * 5,
        out_specs=pl.BlockSpec(memory_space=pltpu.VMEM),
        scratch_shapes=[
            pltpu.VMEM((B, S, HD), jnp.bfloat16),
            pltpu.VMEM((B, S, HD), jnp.bfloat16),
            pltpu.VMEM((N_DEV - 1, B, S, HD), jnp.bfloat16),
            pltpu.VMEM((N_DEV - 1, B, S, HD), jnp.bfloat16),
            pltpu.VMEM((B, S, HD), jnp.bfloat16),
            pltpu.SemaphoreType.DMA((N_DEV - 1, 2)),
            pltpu.SemaphoreType.DMA((N_DEV - 1, 2)),
            pltpu.SemaphoreType.DMA((N_DEV - 1, 2)),
            pltpu.SemaphoreType.DMA((N_DEV - 1, 2)),
        ],
        compiler_params=pltpu.CompilerParams(collective_id=0),
    )(x, Wq, K2, V2, Wo)
    return out


# device time: 49298 ns/iter; 2.2005x vs baseline; 1.0029x over previous
import jax
import jax.numpy as jnp
from jax import lax
from jax.experimental import pallas as pl
from jax.experimental.pallas import tpu as pltpu

N_DEV = 4
BLK = 64
STRIDE = 4


def kernel(x, Wq, K_ext, V_ext, Wo):
    B, S, D = x.shape
    Hq, Dh = K_ext.shape[2], K_ext.shape[3]
    HD = Hq * Dh

    K2 = K_ext.reshape(B, S, HD)
    V2 = V_ext.reshape(B, S, HD)

    def body(x_ref, wq_ref, k_ref, v_ref, wo_ref, out_ref,
             k_loc, v_loc, k_bufs, v_bufs, q_sc,
             ksend, krecv, vsend, vrecv):
        my = lax.axis_index("i")
        left = lax.rem(my + N_DEV - 1, N_DEV)
        right = lax.rem(my + 1, N_DEV)

        k_loc[...] = k_ref[...].astype(jnp.bfloat16)

        barrier = pltpu.get_barrier_semaphore()
        for nbr in (left, right):
            pl.semaphore_signal(barrier, inc=1, device_id=(nbr,),
                                device_id_type=pl.DeviceIdType.MESH)
        pl.semaphore_wait(barrier, 2)

        def make_rdmas(loc, bufs, ssem, rsem, slot, dirs, src_is_input):
            rds = []
            for d, tgt in dirs:
                src = loc.at[d] if src_is_input else bufs.at[slot - 1, d]
                rds.append(pltpu.make_async_remote_copy(
                    src_ref=src, dst_ref=bufs.at[slot, d],
                    send_sem=ssem.at[slot, d], recv_sem=rsem.at[slot, d],
                    device_id=(tgt,),
                    device_id_type=pl.DeviceIdType.MESH))
            return rds

        RING = ((0, right), (1, left))
        REV = ((0, left), (1, right))

        def k_rdmas(slot, dirs, src_is_input=False):
            return make_rdmas(k_loc, k_bufs, ksend, krecv, slot, dirs,
                              src_is_input)

        def v_rdmas(slot, dirs, src_is_input=False):
            return make_rdmas(v_loc, v_bufs, vsend, vrecv, slot, dirs,
                              src_is_input)

        hop0_k = k_rdmas(0, RING, True)
        direct_k = k_rdmas(2, REV, True)
        for r in hop0_k + direct_k:
            r.start()
        v_loc[...] = v_ref[...].astype(jnp.bfloat16)
        hop0_v = v_rdmas(0, RING, True)
        direct_v = v_rdmas(2, REV, True)
        for r in hop0_v + direct_v:
            r.start()

        for b in range(B):
            q_sc[b] = lax.dot_general(
                x_ref[b].astype(jnp.bfloat16),
                wq_ref[...].astype(jnp.bfloat16),
                (((1,), (0,)), ((), ())),
                preferred_element_type=jnp.float32).astype(jnp.bfloat16)

        rows = lax.broadcasted_iota(jnp.int32, (S, S), 0)
        cols = lax.broadcasted_iota(jnp.int32, (S, S), 1)
        mask = ((rows // BLK) % STRIDE) == ((cols // BLK) % STRIDE)

        accs = [[jnp.zeros((S, Dh), jnp.float32) for _ in range(Hq)]
                for _ in range(B)]
        lsums = [[jnp.zeros((S, 1), jnp.float32) for _ in range(Hq)]
                 for _ in range(B)]

        def weights(slot, b, h):
            sl = slice(h * Dh, (h + 1) * Dh)
            qbh = q_sc[b, :, sl]
            kc = k_loc[b, :, sl] if slot is None else k_bufs[slot, b, :, sl]
            s = lax.dot_general(
                qbh, kc, (((1,), (1,)), ((), ())),
                preferred_element_type=jnp.float32) * 0.125
            return jnp.where(mask, jnp.exp(s), 0.0)

        def pv(slot, b, h, w):
            sl = slice(h * Dh, (h + 1) * Dh)
            vc = v_loc[b, :, sl] if slot is None else v_bufs[slot, b, :, sl]
            accs[b][h] = accs[b][h] + lax.dot_general(
                w.astype(jnp.bfloat16), vc, (((1,), (0,)), ((), ())),
                preferred_element_type=jnp.float32)
            lsums[b][h] = lsums[b][h] + jnp.sum(w, axis=1, keepdims=True)

        def accumulate(slot):
            for b in range(B):
                for h in range(Hq):
                    pv(slot, b, h, weights(slot, b, h))

        def finalize_and_project(hs, first):
            lo, hi = hs[0] * Dh, (hs[-1] + 1) * Dh
            for b in range(B):
                for h in hs:
                    sl = slice(h * Dh, (h + 1) * Dh)
                    q_sc[b, :, sl] = (
                        accs[b][h] / lsums[b][h]).astype(jnp.bfloat16)
            wo_bf = wo_ref[lo:hi, :].astype(jnp.bfloat16)
            for b in range(B):
                part = lax.dot_general(
                    q_sc[b, :, lo:hi], wo_bf, (((1,), (0,)), ((), ())),
                    preferred_element_type=jnp.float32).astype(jnp.bfloat16)
                out_ref[b] = part if first else out_ref[b] + part

        accumulate(None)

        for r in hop0_k + hop0_v:
            r.wait_recv()
        H2 = HD // 2
        halves = []
        for idx, (lo, semslot) in enumerate(((0, 1), (H2, 3))):
            cs = slice(lo, lo + H2)
            k_h = [pltpu.make_async_remote_copy(
                src_ref=k_bufs.at[0, d, :, cs], dst_ref=k_bufs.at[1, d, :, cs],
                send_sem=ksend.at[semslot, d], recv_sem=krecv.at[semslot, d],
                device_id=(tgt,), device_id_type=pl.DeviceIdType.MESH)
                for d, tgt in RING]
            v_h = [pltpu.make_async_remote_copy(
                src_ref=v_bufs.at[0, d, :, cs], dst_ref=v_bufs.at[1, d, :, cs],
                send_sem=vsend.at[semslot, d], recv_sem=vrecv.at[semslot, d],
                device_id=(tgt,), device_id_type=pl.DeviceIdType.MESH)
                for d, tgt in RING]
            heads = list(range(idx * Hq // 2, (idx + 1) * Hq // 2))
            halves.append((k_h, v_h, heads))
        for k_h, v_h, _ in halves:
            for r in k_h + v_h:
                r.start()
        accumulate(0)

        for r in direct_k + direct_v:
            r.wait_recv()
        accumulate(2)

        for idx, (k_h, v_h, heads) in enumerate(halves):
            for r in k_h:
                r.wait_recv()
            ws = [[weights(1, b, h) for h in heads] for b in range(B)]
            for r in v_h:
                r.wait_recv()
            for b in range(B):
                for j, h in enumerate(heads):
                    pv(1, b, h, ws[b][j])
            finalize_and_project(heads, first=(idx == 0))

        all_rds = hop0_k + hop0_v + direct_k + direct_v
        for k_h, v_h, _ in halves:
            all_rds += k_h + v_h
        for r in all_rds:
            r.wait_send()

    out = pl.pallas_call(
        body,
        out_shape=jax.ShapeDtypeStruct((B, S, D), jnp.bfloat16),
        in_specs=[pl.BlockSpec(memory_space=pltpu.VMEM)] * 5,
        out_specs=pl.BlockSpec(memory_space=pltpu.VMEM),
        scratch_shapes=[
            pltpu.VMEM((B, S, HD), jnp.bfloat16),
            pltpu.VMEM((B, S, HD), jnp.bfloat16),
            pltpu.VMEM((N_DEV - 1, B, S, HD), jnp.bfloat16),
            pltpu.VMEM((N_DEV - 1, B, S, HD), jnp.bfloat16),
            pltpu.VMEM((B, S, HD), jnp.bfloat16),
            pltpu.SemaphoreType.DMA((N_DEV, 2)),
            pltpu.SemaphoreType.DMA((N_DEV, 2)),
            pltpu.SemaphoreType.DMA((N_DEV, 2)),
            pltpu.SemaphoreType.DMA((N_DEV, 2)),
        ],
        compiler_params=pltpu.CompilerParams(collective_id=0),
    )(x, Wq, K2, V2, Wo)
    return out


# device time: 49067 ns/iter; 2.2108x vs baseline; 1.0047x over previous
import jax
import jax.numpy as jnp
from jax import lax
from jax.experimental import pallas as pl
from jax.experimental.pallas import tpu as pltpu

N_DEV = 4
BLK = 64
STRIDE = 4


def kernel(x, Wq, K_ext, V_ext, Wo):
    B, S, D = x.shape
    Hq, Dh = K_ext.shape[2], K_ext.shape[3]
    HD = Hq * Dh

    K2 = K_ext.reshape(B, S, HD)
    V2 = V_ext.reshape(B, S, HD)

    def body(x_ref, wq_ref, k_ref, v_ref, wo_ref, out_ref,
             k_loc, v_loc, k_bufs, v_bufs, q_sc,
             ksend, krecv, vsend, vrecv):
        my = lax.axis_index("i")
        left = lax.rem(my + N_DEV - 1, N_DEV)
        right = lax.rem(my + 1, N_DEV)

        barrier = pltpu.get_barrier_semaphore()
        for nbr in (left, right):
            pl.semaphore_signal(barrier, inc=1, device_id=(nbr,),
                                device_id_type=pl.DeviceIdType.MESH)
        k_loc[...] = k_ref[...].astype(jnp.bfloat16)
        pl.semaphore_wait(barrier, 2)

        def make_rdmas(loc, bufs, ssem, rsem, slot, dirs, src_is_input):
            rds = []
            for d, tgt in dirs:
                src = loc.at[d] if src_is_input else bufs.at[slot - 1, d]
                rds.append(pltpu.make_async_remote_copy(
                    src_ref=src, dst_ref=bufs.at[slot, d],
                    send_sem=ssem.at[slot, d], recv_sem=rsem.at[slot, d],
                    device_id=(tgt,),
                    device_id_type=pl.DeviceIdType.MESH))
            return rds

        RING = ((0, right), (1, left))
        REV = ((0, left), (1, right))

        def k_rdmas(slot, dirs, src_is_input=False):
            return make_rdmas(k_loc, k_bufs, ksend, krecv, slot, dirs,
                              src_is_input)

        def v_rdmas(slot, dirs, src_is_input=False):
            return make_rdmas(v_loc, v_bufs, vsend, vrecv, slot, dirs,
                              src_is_input)

        hop0_k = k_rdmas(0, RING, True)
        direct_k = k_rdmas(2, REV, True)
        for r in hop0_k + direct_k:
            r.start()
        v_loc[...] = v_ref[...].astype(jnp.bfloat16)
        hop0_v = v_rdmas(0, RING, True)
        direct_v = v_rdmas(2, REV, True)
        for r in hop0_v + direct_v:
            r.start()

        for b in range(B):
            q_sc[b] = lax.dot_general(
                x_ref[b].astype(jnp.bfloat16),
                wq_ref[...].astype(jnp.bfloat16),
                (((1,), (0,)), ((), ())),
                preferred_element_type=jnp.float32).astype(jnp.bfloat16)

        rows = lax.broadcasted_iota(jnp.int32, (S, S), 0)
        cols = lax.broadcasted_iota(jnp.int32, (S, S), 1)
        mask = ((rows // BLK) % STRIDE) == ((cols // BLK) % STRIDE)

        accs = [[jnp.zeros((S, Dh), jnp.float32) for _ in range(Hq)]
                for _ in range(B)]
        lsums = [[jnp.zeros((S, 1), jnp.float32) for _ in range(Hq)]
                 for _ in range(B)]

        def weights(slot, b, h):
            sl = slice(h * Dh, (h + 1) * Dh)
            qbh = q_sc[b, :, sl]
            kc = k_loc[b, :, sl] if slot is None else k_bufs[slot, b, :, sl]
            s = lax.dot_general(
                qbh, kc, (((1,), (1,)), ((), ())),
                preferred_element_type=jnp.float32) * 0.125
            return jnp.where(mask, jnp.exp(s), 0.0)

        def pv(slot, b, h, w):
            sl = slice(h * Dh, (h + 1) * Dh)
            vc = v_loc[b, :, sl] if slot is None else v_bufs[slot, b, :, sl]
            accs[b][h] = accs[b][h] + lax.dot_general(
                w.astype(jnp.bfloat16), vc, (((1,), (0,)), ((), ())),
                preferred_element_type=jnp.float32)
            lsums[b][h] = lsums[b][h] + jnp.sum(w, axis=1, keepdims=True)

        def accumulate(slot):
            for b in range(B):
                for h in range(Hq):
                    pv(slot, b, h, weights(slot, b, h))

        def finalize_and_project(hs, first):
            lo, hi = hs[0] * Dh, (hs[-1] + 1) * Dh
            for b in range(B):
                for h in hs:
                    sl = slice(h * Dh, (h + 1) * Dh)
                    q_sc[b, :, sl] = (
                        accs[b][h] / lsums[b][h]).astype(jnp.bfloat16)
            wo_bf = wo_ref[lo:hi, :].astype(jnp.bfloat16)
            for b in range(B):
                part = lax.dot_general(
                    q_sc[b, :, lo:hi], wo_bf, (((1,), (0,)), ((), ())),
                    preferred_element_type=jnp.float32).astype(jnp.bfloat16)
                out_ref[b] = part if first else out_ref[b] + part

        accumulate(None)

        for r in hop0_k + hop0_v:
            r.wait_recv()
        H2 = HD // 2
        halves = []
        for idx, (lo, semslot) in enumerate(((0, 1), (H2, 3))):
            cs = slice(lo, lo + H2)
            k_h = [pltpu.make_async_remote_copy(
                src_ref=k_bufs.at[0, d, :, cs], dst_ref=k_bufs.at[1, d, :, cs],
                send_sem=ksend.at[semslot, d], recv_sem=krecv.at[semslot, d],
                device_id=(tgt,), device_id_type=pl.DeviceIdType.MESH)
                for d, tgt in RING]
            v_h = [pltpu.make_async_remote_copy(
                src_ref=v_bufs.at[0, d, :, cs], dst_ref=v_bufs.at[1, d, :, cs],
                send_sem=vsend.at[semslot, d], recv_sem=vrecv.at[semslot, d],
                device_id=(tgt,), device_id_type=pl.DeviceIdType.MESH)
                for d, tgt in RING]
            heads = list(range(idx * Hq // 2, (idx + 1) * Hq // 2))
            halves.append((k_h, v_h, heads))
        for k_h, v_h, _ in halves:
            for r in k_h + v_h:
                r.start()
        accumulate(0)

        for r in direct_k + direct_v:
            r.wait_recv()
        accumulate(2)

        for idx, (k_h, v_h, heads) in enumerate(halves):
            for r in k_h:
                r.wait_recv()
            ws = [[weights(1, b, h) for h in heads] for b in range(B)]
            for r in v_h:
                r.wait_recv()
            for b in range(B):
                for j, h in enumerate(heads):
                    pv(1, b, h, ws[b][j])
            finalize_and_project(heads, first=(idx == 0))

        all_rds = hop0_k + hop0_v + direct_k + direct_v
        for k_h, v_h, _ in halves:
            all_rds += k_h + v_h
        for r in all_rds:
            r.wait_send()

    out = pl.pallas_call(
        body,
        out_shape=jax.ShapeDtypeStruct((B, S, D), jnp.bfloat16),
        in_specs=[pl.BlockSpec(memory_space=pltpu.VMEM)] * 5,
        out_specs=pl.BlockSpec(memory_space=pltpu.VMEM),
        scratch_shapes=[
            pltpu.VMEM((B, S, HD), jnp.bfloat16),
            pltpu.VMEM((B, S, HD), jnp.bfloat16),
            pltpu.VMEM((N_DEV - 1, B, S, HD), jnp.bfloat16),
            pltpu.VMEM((N_DEV - 1, B, S, HD), jnp.bfloat16),
            pltpu.VMEM((B, S, HD), jnp.bfloat16),
            pltpu.SemaphoreType.DMA((N_DEV, 2)),
            pltpu.SemaphoreType.DMA((N_DEV, 2)),
            pltpu.SemaphoreType.DMA((N_DEV, 2)),
            pltpu.SemaphoreType.DMA((N_DEV, 2)),
        ],
        compiler_params=pltpu.CompilerParams(collective_id=0),
    )(x, Wq, K2, V2, Wo)
    return out
